# baseline (speedup 1.0000x reference)
import jax
import jax.numpy as jnp
from jax.experimental import pallas as pl

N = 10000
E = 320000
D = 128
H = 16

EBLK = 6400
NEB = E // EBLK


def _qproj_body(atom_ref, wq_ref, q_ref):
    q_ref[...] = jnp.dot(atom_ref[...], wq_ref[...],
                         preferred_element_type=jnp.float32)


def _edge_fwd_body(qs_ref, xd_ref, ef_ref, sh_ref,
                   wk1_ref, bk1_ref, wk2_ref, bk2_ref,
                   wv1_ref, bv1_ref, wv2_ref, bv2_ref,
                   attn_ref, v_ref):
    ef = ef_ref[...]
    kh = jnp.maximum(jnp.dot(ef, wk1_ref[...],
                             preferred_element_type=jnp.float32)
                     + bk1_ref[...], 0.0)
    kw = jnp.dot(kh, wk2_ref[...],
                 preferred_element_type=jnp.float32) + bk2_ref[...]
    vh = jnp.maximum(jnp.dot(ef, wv1_ref[...],
                             preferred_element_type=jnp.float32)
                     + bv1_ref[...], 0.0)
    vw = jnp.dot(vh, wv2_ref[...],
                 preferred_element_type=jnp.float32) + bv2_ref[...]
    xd = xd_ref[...]
    sh = sh_ref[...]
    v_ref[...] = xd * sh * vw
    attn_ref[...] = jnp.sum(qs_ref[...] * xd * kw, axis=1, keepdims=True) * sh


def _node_update_body(atom_ref, upd_ref, gamma_ref, beta_ref, we_ref,
                      x_ref, sa_ref):
    x = atom_ref[...] + upd_ref[...]
    mu = jnp.mean(x, axis=0, keepdims=True)
    xc = x - mu
    var = jnp.mean(xc * xc, axis=0, keepdims=True)
    xn = xc * jax.lax.rsqrt(var + 1e-5) * gamma_ref[...] + beta_ref[...]
    x_ref[...] = xn
    sa_ref[...] = jnp.dot(xn, we_ref[...], preferred_element_type=jnp.float32)


def _edge_update_body(sd_ref, ss_ref, ef_ref,
                      we1a_ref, we1b_ref, we1c_ref, be1_ref,
                      we2_ref, be2_ref, we3_ref, be3_ref,
                      lng_ref, lnb_ref, e_ref):
    ef = ef_ref[...]
    h = (jnp.dot(sd_ref[...], we1a_ref[...], preferred_element_type=jnp.float32)
         + jnp.dot(ss_ref[...], we1b_ref[...], preferred_element_type=jnp.float32)
         + jnp.dot(ef, we1c_ref[...], preferred_element_type=jnp.float32)
         + be1_ref[...])
    h = jnp.maximum(h, 0.0)
    h = jnp.maximum(jnp.dot(h, we2_ref[...],
                            preferred_element_type=jnp.float32)
                    + be2_ref[...], 0.0)
    h = jnp.dot(h, we3_ref[...], preferred_element_type=jnp.float32) + be3_ref[...]
    e = ef + h
    mu = jnp.mean(e, axis=1, keepdims=True)
    ec = e - mu
    var = jnp.mean(ec * ec, axis=1, keepdims=True)
    e_ref[...] = ec * jax.lax.rsqrt(var + 1e-5) * lng_ref[...] + lnb_ref[...]


def _full_spec(shape):
    return pl.BlockSpec(shape, lambda *_: tuple(0 for _ in shape))


def kernel(atom_features, edge_features, edge_sh, edge_index,
           W_q, Wk1, bk1, Wk2, bk2, Wv1, bv1, Wv2, bv2,
           bn_gamma, bn_beta, W_e, We1, be1, We2, be2, We3, be3,
           ln_gamma, ln_beta, interpret=False):
    dst = edge_index[0]
    src = edge_index[1]

    qfull = pl.pallas_call(
        _qproj_body,
        out_shape=jax.ShapeDtypeStruct((N, D), jnp.float32),
        interpret=interpret,
    )(atom_features, W_q)

    qs = jnp.take(qfull, src, axis=0)
    xd = jnp.take(atom_features, dst, axis=0)

    eb = lambda i: (i, 0)
    attn2, v = pl.pallas_call(
        _edge_fwd_body,
        grid=(NEB,),
        in_specs=[
            pl.BlockSpec((EBLK, D), eb),
            pl.BlockSpec((EBLK, D), eb),
            pl.BlockSpec((EBLK, H), eb),
            pl.BlockSpec((EBLK, 1), eb),
            _full_spec((H, H)), _full_spec((1, H)),
            _full_spec((H, D)), _full_spec((1, D)),
            _full_spec((H, H)), _full_spec((1, H)),
            _full_spec((H, D)), _full_spec((1, D)),
        ],
        out_specs=[
            pl.BlockSpec((EBLK, 1), eb),
            pl.BlockSpec((EBLK, D), eb),
        ],
        out_shape=[
            jax.ShapeDtypeStruct((E, 1), jnp.float32),
            jax.ShapeDtypeStruct((E, D), jnp.float32),
        ],
        interpret=interpret,
    )(qs, xd, edge_features, edge_sh,
      Wk1, bk1.reshape(1, H), Wk2, bk2.reshape(1, D),
      Wv1, bv1.reshape(1, H), Wv2, bv2.reshape(1, D))
    attn = attn2[:, 0]

    m = jax.ops.segment_max(attn, src, num_segments=N)
    m = jnp.where(jnp.isfinite(m), m, 0.0)
    ex = jnp.exp(attn - m[src])
    s = jax.ops.segment_sum(ex, src, num_segments=N)
    alpha = ex / (s[src] + 1e-16)

    upd = jax.ops.segment_sum(v * alpha[:, None], src, num_segments=N)

    x, sa = pl.pallas_call(
        _node_update_body,
        out_shape=[
            jax.ShapeDtypeStruct((N, D), jnp.float32),
            jax.ShapeDtypeStruct((N, H), jnp.float32),
        ],
        interpret=interpret,
    )(atom_features, upd, bn_gamma.reshape(1, D), bn_beta.reshape(1, D), W_e)

    sd = jnp.take(sa, dst, axis=0)
    ss = jnp.take(sa, src, axis=0)

    e = pl.pallas_call(
        _edge_update_body,
        grid=(NEB,),
        in_specs=[
            pl.BlockSpec((EBLK, H), eb),
            pl.BlockSpec((EBLK, H), eb),
            pl.BlockSpec((EBLK, H), eb),
            _full_spec((H, H)), _full_spec((H, H)), _full_spec((H, H)),
            _full_spec((1, H)),
            _full_spec((H, H)), _full_spec((1, H)),
            _full_spec((H, H)), _full_spec((1, H)),
            _full_spec((1, H)), _full_spec((1, H)),
        ],
        out_specs=pl.BlockSpec((EBLK, H), eb),
        out_shape=jax.ShapeDtypeStruct((E, H), jnp.float32),
        interpret=interpret,
    )(sd, ss, edge_features,
      We1[0:H], We1[H:2 * H], We1[2 * H:3 * H], be1.reshape(1, H),
      We2, be2.reshape(1, H), We3, be3.reshape(1, H),
      ln_gamma.reshape(1, H), ln_beta.reshape(1, H))

    return (x, e)


# speedup vs baseline: 4.3718x; 4.3718x over previous
import functools

import jax
import jax.numpy as jnp
from jax import lax
from jax.experimental import pallas as pl
from jax.experimental.pallas import tpu as pltpu
from jax.experimental.pallas import tpu_sc as plsc

N = 10000
E = 320000
D = 128
H = 16

EBLK = 6400
NEB = E // EBLK

SC_NC = 2
SC_NS = 16
SC_NW = SC_NC * SC_NS
GCHUNK = 400
EPW = E // SC_NW
NSTEP = EPW // GCHUNK


def _sc_mesh():
    return plsc.VectorSubcoreMesh(core_axis_name="c", subcore_axis_name="s")


_SC_CP = pltpu.CompilerParams(needs_layout_passes=False)


def _gather2_body(qtab_hbm, xtab_hbm, src_hbm, dst_hbm, qs_hbm, xd_hbm,
                  sidx_v, didx_v, qrows_v, xrows_v, sem_q, sem_x):
    wid = lax.axis_index("s") * SC_NC + lax.axis_index("c")
    base0 = wid * EPW

    @pl.loop(0, NSTEP)
    def _(c):
        base = base0 + c * GCHUNK
        pltpu.sync_copy(src_hbm.at[pl.ds(base, GCHUNK)], sidx_v)
        pltpu.sync_copy(dst_hbm.at[pl.ds(base, GCHUNK)], didx_v)
        cp_q = pltpu.async_copy(qtab_hbm.at[sidx_v], qrows_v, sem_q)
        cp_x = pltpu.async_copy(xtab_hbm.at[didx_v], xrows_v, sem_x)
        cp_q.wait()
        cp_x.wait()
        pltpu.sync_copy(qrows_v, qs_hbm.at[pl.ds(base, GCHUNK)])
        pltpu.sync_copy(xrows_v, xd_hbm.at[pl.ds(base, GCHUNK)])


def _sc_gather2(qtab, xtab, src, dst):
    f = pl.kernel(
        _gather2_body,
        out_type=[
            jax.ShapeDtypeStruct((E, D), jnp.float32),
            jax.ShapeDtypeStruct((E, D), jnp.float32),
        ],
        mesh=_sc_mesh(),
        scratch_types=[
            pltpu.VMEM((GCHUNK,), jnp.int32),
            pltpu.VMEM((GCHUNK,), jnp.int32),
            pltpu.VMEM((GCHUNK, D), jnp.float32),
            pltpu.VMEM((GCHUNK, D), jnp.float32),
            pltpu.SemaphoreType.DMA,
            pltpu.SemaphoreType.DMA,
        ],
    )
    return f(qtab, xtab, src, dst)


NPAD = 10240
EPT = E // SC_NS
NROW = NPAD // SC_NS


def _softmax_body(attn_hbm, src_hbm, alpha_hbm,
                  m_v, s_v, attn_v, idx_v, ex_v, tmp_v, acc_v,
                  stage_sh, final_sh):
    cid = lax.axis_index("c")
    tid = lax.axis_index("s")
    ebase = tid * EPT
    rbase = tid * NROW

    @pl.when(cid == 0)
    def _():
        @pl.loop(0, NPAD, step=16)
        def _(i):
            m_v[pl.ds(i, 16)] = jnp.full((16,), -3e38, jnp.float32)

        pltpu.sync_copy(attn_hbm.at[pl.ds(ebase, EPT)], attn_v)
        pltpu.sync_copy(src_hbm.at[pl.ds(ebase, EPT)], idx_v)

        @pl.loop(0, EPT, step=16)
        def _(i):
            iv = idx_v[pl.ds(i, 16)]
            av = attn_v[pl.ds(i, 16)]
            mg = plsc.load_gather(m_v, [iv])
            plsc.store_scatter(m_v, [iv], jnp.maximum(mg, av))

        pltpu.sync_copy(m_v, stage_sh.at[tid])

    plsc.subcore_barrier()

    @pl.when(cid == 0)
    def _():
        pltpu.sync_copy(stage_sh.at[0, pl.ds(rbase, NROW)], acc_v)

        @pl.loop(1, SC_NS)
        def _(j):
            pltpu.sync_copy(stage_sh.at[j, pl.ds(rbase, NROW)], tmp_v)

            @pl.loop(0, NROW, step=16)
            def _(i):
                acc_v[pl.ds(i, 16)] = jnp.maximum(acc_v[pl.ds(i, 16)],
                                                  tmp_v[pl.ds(i, 16)])

        pltpu.sync_copy(acc_v, final_sh.at[pl.ds(rbase, NROW)])

    plsc.subcore_barrier()

    @pl.when(cid == 0)
    def _():
        pltpu.sync_copy(final_sh, m_v)

        @pl.loop(0, NPAD, step=16)
        def _(i):
            s_v[pl.ds(i, 16)] = jnp.zeros((16,), jnp.float32)

        @pl.loop(0, EPT, step=16)
        def _(i):
            iv = idx_v[pl.ds(i, 16)]
            av = attn_v[pl.ds(i, 16)]
            mg = plsc.load_gather(m_v, [iv])
            ex = jnp.exp(av - mg)
            ex_v[pl.ds(i, 16)] = ex
            plsc.addupdate_scatter(s_v, [iv], ex)

        pltpu.sync_copy(s_v, stage_sh.at[tid])

    plsc.subcore_barrier()

    @pl.when(cid == 0)
    def _():
        pltpu.sync_copy(stage_sh.at[0, pl.ds(rbase, NROW)], acc_v)

        @pl.loop(1, SC_NS)
        def _(j):
            pltpu.sync_copy(stage_sh.at[j, pl.ds(rbase, NROW)], tmp_v)

            @pl.loop(0, NROW, step=16)
            def _(i):
                acc_v[pl.ds(i, 16)] = acc_v[pl.ds(i, 16)] + tmp_v[pl.ds(i, 16)]

        @pl.loop(0, NROW, step=16)
        def _(i):
            acc_v[pl.ds(i, 16)] = 1.0 / (acc_v[pl.ds(i, 16)] + 1e-16)

        pltpu.sync_copy(acc_v, final_sh.at[pl.ds(rbase, NROW)])

    plsc.subcore_barrier()

    @pl.when(cid == 0)
    def _():
        pltpu.sync_copy(final_sh, m_v)

        @pl.loop(0, EPT, step=16)
        def _(i):
            iv = idx_v[pl.ds(i, 16)]
            rg = plsc.load_gather(m_v, [iv])
            ex_v[pl.ds(i, 16)] = ex_v[pl.ds(i, 16)] * rg

        pltpu.sync_copy(ex_v, alpha_hbm.at[pl.ds(ebase, EPT)])


def _sc_softmax(attn, src):
    f = pl.kernel(
        _softmax_body,
        out_type=jax.ShapeDtypeStruct((E,), jnp.float32),
        mesh=_sc_mesh(),
        scratch_types=[
            pltpu.VMEM((NPAD,), jnp.float32),
            pltpu.VMEM((NPAD,), jnp.float32),
            pltpu.VMEM((EPT,), jnp.float32),
            pltpu.VMEM((EPT,), jnp.int32),
            pltpu.VMEM((EPT,), jnp.float32),
            pltpu.VMEM((NROW,), jnp.float32),
            pltpu.VMEM((NROW,), jnp.float32),
            pltpu.VMEM_SHARED((SC_NS, NPAD), jnp.float32),
            pltpu.VMEM_SHARED((NPAD,), jnp.float32),
        ],
        compiler_params=_SC_CP,
    )
    return f(attn, src)


NH = N // SC_NC
NACC = NH + 8
EPT_S = E // SC_NS
NSTEP_S = EPT_S // GCHUNK
WCH = 80
NW_SUB = GCHUNK // WCH


def _scatter_body(va_hbm, src_hbm, zeros_hbm, out_hbm,
                  idx_v, idx2_v, rows_v, acc_sh):
    cid = lax.axis_index("c")
    tid = lax.axis_index("s")
    base0 = tid * EPT_S
    noff = cid * NH

    @pl.when(tid == 0)
    def _():
        pltpu.sync_copy(zeros_hbm, acc_sh)

    plsc.subcore_barrier()

    @pl.loop(0, NSTEP_S)
    def _(c):
        base = base0 + c * GCHUNK
        pltpu.sync_copy(src_hbm.at[pl.ds(base, GCHUNK)], idx_v)
        pltpu.sync_copy(va_hbm.at[pl.ds(base, GCHUNK)], rows_v)

        for j in range(NW_SUB):
            for k in range(0, WCH, 16):
                t = idx_v[pl.ds(j * WCH + k, 16)] - noff
                oob = (t < 0) | (t >= NH)
                idx2_v[j, pl.ds(k, 16)] = jnp.where(oob, NH, t)

        for j in range(NW_SUB):
            pltpu.sync_copy(rows_v.at[pl.ds(j * WCH, WCH)],
                            acc_sh.at[idx2_v.at[j]], add=True)

    plsc.subcore_barrier()

    @pl.when(tid < 5)
    def _():
        pltpu.sync_copy(acc_sh.at[pl.ds(tid * 1000, 1000)],
                        out_hbm.at[cid, pl.ds(tid * 1000, 1000)])


def _sc_scatter(va, src, zeros_nd):
    f = pl.kernel(
        _scatter_body,
        out_type=jax.ShapeDtypeStruct((SC_NC, NH, D), jnp.float32),
        mesh=_sc_mesh(),
        scratch_types=[
            pltpu.VMEM((GCHUNK,), jnp.int32),
            pltpu.VMEM((NW_SUB, WCH), jnp.int32),
            pltpu.VMEM((GCHUNK, D), jnp.float32),
            pltpu.VMEM_SHARED((NACC, D), jnp.float32),
        ],
        compiler_params=_SC_CP,
    )
    return f(va, src, zeros_nd)


def _qproj_body(atom_ref, wq_ref, q_ref):
    q_ref[...] = jnp.dot(atom_ref[...], wq_ref[...],
                         preferred_element_type=jnp.float32)


def _edge_attn_body(qs_ref, xd_ref, ef_ref, sh_ref,
                    wk1_ref, bk1_ref, wk2_ref, bk2_ref, attn_ref):
    ef = ef_ref[...]
    kh = jnp.maximum(jnp.dot(ef, wk1_ref[...],
                             preferred_element_type=jnp.float32)
                     + bk1_ref[...], 0.0)
    kw = jnp.dot(kh, wk2_ref[...],
                 preferred_element_type=jnp.float32) + bk2_ref[...]
    attn_ref[...] = jnp.sum(qs_ref[...] * xd_ref[...] * kw,
                            axis=1, keepdims=True) * sh_ref[...]


def _edge_va_body(xd_ref, ef_ref, sh_ref, al_ref,
                  wv1_ref, bv1_ref, wv2_ref, bv2_ref, va_ref):
    ef = ef_ref[...]
    vh = jnp.maximum(jnp.dot(ef, wv1_ref[...],
                             preferred_element_type=jnp.float32)
                     + bv1_ref[...], 0.0)
    vw = jnp.dot(vh, wv2_ref[...],
                 preferred_element_type=jnp.float32) + bv2_ref[...]
    va_ref[...] = xd_ref[...] * (sh_ref[...] * al_ref[...]) * vw


def _node_update_body(atom_ref, ulo_ref, uhi_ref, gamma_ref, beta_ref, we_ref,
                      x_ref, sa_ref):
    x = atom_ref[...] + jnp.concatenate([ulo_ref[...], uhi_ref[...]], axis=0)
    mu = jnp.mean(x, axis=0, keepdims=True)
    xc = x - mu
    var = jnp.mean(xc * xc, axis=0, keepdims=True)
    xn = xc * jax.lax.rsqrt(var + 1e-5) * gamma_ref[...] + beta_ref[...]
    x_ref[...] = xn
    sa = jnp.dot(xn, we_ref[...], preferred_element_type=jnp.float32)
    sa_ref[...] = jnp.concatenate(
        [sa, jnp.zeros((sa.shape[0], D - H), jnp.float32)], axis=1)


def _edge_update_body(sd_ref, ss_ref, ef_ref,
                      we1a_ref, we1b_ref, we1c_ref, be1_ref,
                      we2_ref, be2_ref, we3_ref, be3_ref,
                      lng_ref, lnb_ref, e_ref):
    ef = ef_ref[...]
    h = (jnp.dot(sd_ref[:, 0:H], we1a_ref[...], preferred_element_type=jnp.float32)
         + jnp.dot(ss_ref[:, 0:H], we1b_ref[...], preferred_element_type=jnp.float32)
         + jnp.dot(ef, we1c_ref[...], preferred_element_type=jnp.float32)
         + be1_ref[...])
    h = jnp.maximum(h, 0.0)
    h = jnp.maximum(jnp.dot(h, we2_ref[...],
                            preferred_element_type=jnp.float32)
                    + be2_ref[...], 0.0)
    h = jnp.dot(h, we3_ref[...], preferred_element_type=jnp.float32) + be3_ref[...]
    e = ef + h
    mu = jnp.mean(e, axis=1, keepdims=True)
    ec = e - mu
    var = jnp.mean(ec * ec, axis=1, keepdims=True)
    e_ref[...] = ec * jax.lax.rsqrt(var + 1e-5) * lng_ref[...] + lnb_ref[...]


def _full_spec(shape):
    return pl.BlockSpec(shape, lambda *_: tuple(0 for _ in shape))


def kernel(atom_features, edge_features, edge_sh, edge_index,
           W_q, Wk1, bk1, Wk2, bk2, Wv1, bv1, Wv2, bv2,
           bn_gamma, bn_beta, W_e, We1, be1, We2, be2, We3, be3,
           ln_gamma, ln_beta, interpret=False):
    dst = edge_index[0]
    src = edge_index[1]

    qfull = pl.pallas_call(
        _qproj_body,
        out_shape=jax.ShapeDtypeStruct((N, D), jnp.float32),
        interpret=interpret,
    )(atom_features, W_q)

    if interpret:
        qs = jnp.take(qfull, src, axis=0)
        xd = jnp.take(atom_features, dst, axis=0)
    else:
        qs, xd = _sc_gather2(qfull, atom_features, src, dst)

    eb = lambda i: (i, 0)
    attn2 = pl.pallas_call(
        _edge_attn_body,
        grid=(NEB,),
        in_specs=[
            pl.BlockSpec((EBLK, D), eb),
            pl.BlockSpec((EBLK, D), eb),
            pl.BlockSpec((EBLK, H), eb),
            pl.BlockSpec((EBLK, 1), eb),
            _full_spec((H, H)), _full_spec((1, H)),
            _full_spec((H, D)), _full_spec((1, D)),
        ],
        out_specs=pl.BlockSpec((EBLK, 1), eb),
        out_shape=jax.ShapeDtypeStruct((E, 1), jnp.float32),
        interpret=interpret,
    )(qs, xd, edge_features, edge_sh,
      Wk1, bk1.reshape(1, H), Wk2, bk2.reshape(1, D))
    attn = attn2[:, 0]

    if interpret:
        m = jax.ops.segment_max(attn, src, num_segments=N)
        m = jnp.where(jnp.isfinite(m), m, 0.0)
        ex = jnp.exp(attn - m[src])
        s = jax.ops.segment_sum(ex, src, num_segments=N)
        alpha = ex / (s[src] + 1e-16)
    else:
        alpha = _sc_softmax(attn, src)

    va = pl.pallas_call(
        _edge_va_body,
        grid=(NEB,),
        in_specs=[
            pl.BlockSpec((EBLK, D), eb),
            pl.BlockSpec((EBLK, H), eb),
            pl.BlockSpec((EBLK, 1), eb),
            pl.BlockSpec((EBLK, 1), eb),
            _full_spec((H, H)), _full_spec((1, H)),
            _full_spec((H, D)), _full_spec((1, D)),
        ],
        out_specs=pl.BlockSpec((EBLK, D), eb),
        out_shape=jax.ShapeDtypeStruct((E, D), jnp.float32),
        interpret=interpret,
    )(xd, edge_features, edge_sh, alpha.reshape(E, 1),
      Wv1, bv1.reshape(1, H), Wv2, bv2.reshape(1, D))

    if interpret:
        upd = jax.ops.segment_sum(va, src, num_segments=N)
        upd_a, upd_b = upd[0:NH], upd[NH:N]
    else:
        zeros_nd = jnp.zeros((NACC, D), jnp.float32)
        updp = _sc_scatter(va, src, zeros_nd)
        upd_a, upd_b = updp[0], updp[1]

    x, sa = pl.pallas_call(
        _node_update_body,
        out_shape=[
            jax.ShapeDtypeStruct((N, D), jnp.float32),
            jax.ShapeDtypeStruct((N, D), jnp.float32),
        ],
        interpret=interpret,
    )(atom_features, upd_a, upd_b,
      bn_gamma.reshape(1, D), bn_beta.reshape(1, D), W_e)

    if interpret:
        sd = jnp.take(sa, dst, axis=0)
        ss = jnp.take(sa, src, axis=0)
    else:
        ss, sd = _sc_gather2(sa, sa, src, dst)

    e = pl.pallas_call(
        _edge_update_body,
        grid=(NEB,),
        in_specs=[
            pl.BlockSpec((EBLK, D), eb),
            pl.BlockSpec((EBLK, D), eb),
            pl.BlockSpec((EBLK, H), eb),
            _full_spec((H, H)), _full_spec((H, H)), _full_spec((H, H)),
            _full_spec((1, H)),
            _full_spec((H, H)), _full_spec((1, H)),
            _full_spec((H, H)), _full_spec((1, H)),
            _full_spec((1, H)), _full_spec((1, H)),
        ],
        out_specs=pl.BlockSpec((EBLK, H), eb),
        out_shape=jax.ShapeDtypeStruct((E, H), jnp.float32),
        interpret=interpret,
    )(sd, ss, edge_features,
      We1[0:H], We1[H:2 * H], We1[2 * H:3 * H], be1.reshape(1, H),
      We2, be2.reshape(1, H), We3, be3.reshape(1, H),
      ln_gamma.reshape(1, H), ln_beta.reshape(1, H))

    return (x, e)


# speedup vs baseline: 4.4587x; 1.0199x over previous
import functools

import jax
import jax.numpy as jnp
from jax import lax
from jax.experimental import pallas as pl
from jax.experimental.pallas import tpu as pltpu
from jax.experimental.pallas import tpu_sc as plsc

N = 10000
E = 320000
D = 128
H = 16

EBLK = 6400
NEB = E // EBLK

SC_NC = 2
SC_NS = 16
SC_NW = SC_NC * SC_NS
GCHUNK = 400
EPW = E // SC_NW
NSTEP = EPW // GCHUNK
GC2 = 200
NSTEP2 = EPW // GC2


def _sc_mesh():
    return plsc.VectorSubcoreMesh(core_axis_name="c", subcore_axis_name="s")


_SC_CP = pltpu.CompilerParams(needs_layout_passes=False)


def _gather2_body(qtab_hbm, xtab_hbm, src_hbm, dst_hbm, qs_hbm, xd_hbm,
                  sidx_v, didx_v, qrows_v, xrows_v, sem_g, sem_w):
    wid = lax.axis_index("s") * SC_NC + lax.axis_index("c")
    base0 = wid * EPW

    pltpu.sync_copy(src_hbm.at[pl.ds(base0, EPW)], sidx_v)
    pltpu.sync_copy(dst_hbm.at[pl.ds(base0, EPW)], didx_v)

    def start_gather(c, b):
        off = c * GC2
        pltpu.async_copy(qtab_hbm.at[sidx_v.at[pl.ds(off, GC2)]],
                         qrows_v.at[b], sem_g.at[b])
        pltpu.async_copy(xtab_hbm.at[didx_v.at[pl.ds(off, GC2)]],
                         xrows_v.at[b], sem_g.at[b])

    def wait_gather(b):
        pltpu.make_async_copy(qtab_hbm.at[sidx_v.at[pl.ds(0, GC2)]],
                              qrows_v.at[b], sem_g.at[b]).wait()
        pltpu.make_async_copy(xtab_hbm.at[didx_v.at[pl.ds(0, GC2)]],
                              xrows_v.at[b], sem_g.at[b]).wait()

    def start_write(c, b):
        base = base0 + c * GC2
        pltpu.async_copy(qrows_v.at[b], qs_hbm.at[pl.ds(base, GC2)],
                         sem_w.at[b])
        pltpu.async_copy(xrows_v.at[b], xd_hbm.at[pl.ds(base, GC2)],
                         sem_w.at[b])

    def wait_write(b):
        pltpu.make_async_copy(qrows_v.at[b],
                              qs_hbm.at[pl.ds(base0, GC2)],
                              sem_w.at[b]).wait()
        pltpu.make_async_copy(xrows_v.at[b],
                              xd_hbm.at[pl.ds(base0, GC2)],
                              sem_w.at[b]).wait()

    @pl.loop(0, NSTEP2, step=2)
    def _(c):
        for b in range(2):
            cc = c + b

            @pl.when(cc >= 2)
            def _():
                wait_write(b)

            start_gather(cc, b)

            @pl.when(cc >= 1)
            def _():
                wait_gather(1 - b)
                start_write(cc - 1, 1 - b)

    last = NSTEP2 - 1
    lb = last % 2
    wait_gather(lb)
    start_write(last, lb)
    wait_write(1 - lb)
    wait_write(lb)


def _sc_gather2(qtab, xtab, src, dst):
    f = pl.kernel(
        _gather2_body,
        out_type=[
            jax.ShapeDtypeStruct((E, D), jnp.float32),
            jax.ShapeDtypeStruct((E, D), jnp.float32),
        ],
        mesh=_sc_mesh(),
        scratch_types=[
            pltpu.VMEM((EPW,), jnp.int32),
            pltpu.VMEM((EPW,), jnp.int32),
            pltpu.VMEM((2, GC2, D), jnp.float32),
            pltpu.VMEM((2, GC2, D), jnp.float32),
            pltpu.SemaphoreType.DMA((2,)),
            pltpu.SemaphoreType.DMA((2,)),
        ],
    )
    return f(qtab, xtab, src, dst)


NPAD = 10240
EPT = E // SC_NS
NROW = NPAD // SC_NS


def _softmax_body(attn_hbm, src_hbm, alpha_hbm,
                  m_v, s_v, attn_v, idx_v, ex_v, tmp_v, acc_v,
                  stage_sh, final_sh):
    cid = lax.axis_index("c")
    tid = lax.axis_index("s")
    ebase = tid * EPT
    rbase = tid * NROW

    @pl.when(cid == 0)
    def _():
        @pl.loop(0, NPAD, step=16)
        def _(i):
            m_v[pl.ds(i, 16)] = jnp.full((16,), -3e38, jnp.float32)

        pltpu.sync_copy(attn_hbm.at[pl.ds(ebase, EPT)], attn_v)
        pltpu.sync_copy(src_hbm.at[pl.ds(ebase, EPT)], idx_v)

        @pl.loop(0, EPT, step=16)
        def _(i):
            iv = idx_v[pl.ds(i, 16)]
            av = attn_v[pl.ds(i, 16)]
            mg = plsc.load_gather(m_v, [iv])
            plsc.store_scatter(m_v, [iv], jnp.maximum(mg, av))

        pltpu.sync_copy(m_v, stage_sh.at[tid])

    plsc.subcore_barrier()

    @pl.when(cid == 0)
    def _():
        pltpu.sync_copy(stage_sh.at[0, pl.ds(rbase, NROW)], acc_v)

        @pl.loop(1, SC_NS)
        def _(j):
            pltpu.sync_copy(stage_sh.at[j, pl.ds(rbase, NROW)], tmp_v)

            @pl.loop(0, NROW, step=16)
            def _(i):
                acc_v[pl.ds(i, 16)] = jnp.maximum(acc_v[pl.ds(i, 16)],
                                                  tmp_v[pl.ds(i, 16)])

        pltpu.sync_copy(acc_v, final_sh.at[pl.ds(rbase, NROW)])

    plsc.subcore_barrier()

    @pl.when(cid == 0)
    def _():
        pltpu.sync_copy(final_sh, m_v)

        @pl.loop(0, NPAD, step=16)
        def _(i):
            s_v[pl.ds(i, 16)] = jnp.zeros((16,), jnp.float32)

        @pl.loop(0, EPT, step=16)
        def _(i):
            iv = idx_v[pl.ds(i, 16)]
            av = attn_v[pl.ds(i, 16)]
            mg = plsc.load_gather(m_v, [iv])
            ex = jnp.exp(av - mg)
            ex_v[pl.ds(i, 16)] = ex
            plsc.addupdate_scatter(s_v, [iv], ex)

        pltpu.sync_copy(s_v, stage_sh.at[tid])

    plsc.subcore_barrier()

    @pl.when(cid == 0)
    def _():
        pltpu.sync_copy(stage_sh.at[0, pl.ds(rbase, NROW)], acc_v)

        @pl.loop(1, SC_NS)
        def _(j):
            pltpu.sync_copy(stage_sh.at[j, pl.ds(rbase, NROW)], tmp_v)

            @pl.loop(0, NROW, step=16)
            def _(i):
                acc_v[pl.ds(i, 16)] = acc_v[pl.ds(i, 16)] + tmp_v[pl.ds(i, 16)]

        @pl.loop(0, NROW, step=16)
        def _(i):
            acc_v[pl.ds(i, 16)] = 1.0 / (acc_v[pl.ds(i, 16)] + 1e-16)

        pltpu.sync_copy(acc_v, final_sh.at[pl.ds(rbase, NROW)])

    plsc.subcore_barrier()

    @pl.when(cid == 0)
    def _():
        pltpu.sync_copy(final_sh, m_v)

        @pl.loop(0, EPT, step=16)
        def _(i):
            iv = idx_v[pl.ds(i, 16)]
            rg = plsc.load_gather(m_v, [iv])
            ex_v[pl.ds(i, 16)] = ex_v[pl.ds(i, 16)] * rg

        pltpu.sync_copy(ex_v, alpha_hbm.at[pl.ds(ebase, EPT)])


def _sc_softmax(attn, src):
    f = pl.kernel(
        _softmax_body,
        out_type=jax.ShapeDtypeStruct((E,), jnp.float32),
        mesh=_sc_mesh(),
        scratch_types=[
            pltpu.VMEM((NPAD,), jnp.float32),
            pltpu.VMEM((NPAD,), jnp.float32),
            pltpu.VMEM((EPT,), jnp.float32),
            pltpu.VMEM((EPT,), jnp.int32),
            pltpu.VMEM((EPT,), jnp.float32),
            pltpu.VMEM((NROW,), jnp.float32),
            pltpu.VMEM((NROW,), jnp.float32),
            pltpu.VMEM_SHARED((SC_NS, NPAD), jnp.float32),
            pltpu.VMEM_SHARED((NPAD,), jnp.float32),
        ],
        compiler_params=_SC_CP,
    )
    return f(attn, src)


NH = N // SC_NC
NACC = NH + 8
EPT_S = E // SC_NS
NSTEP_S = EPT_S // GCHUNK
WCH = 80
NW_SUB = GCHUNK // WCH


def _scatter_body(va_hbm, src_hbm, zeros_hbm, out_hbm,
                  idx_v, idx2_v, rows_v, acc_sh):
    cid = lax.axis_index("c")
    tid = lax.axis_index("s")
    base0 = tid * EPT_S
    noff = cid * NH

    @pl.when(tid == 0)
    def _():
        pltpu.sync_copy(zeros_hbm, acc_sh)

    plsc.subcore_barrier()

    @pl.loop(0, NSTEP_S)
    def _(c):
        base = base0 + c * GCHUNK
        pltpu.sync_copy(src_hbm.at[pl.ds(base, GCHUNK)], idx_v)
        pltpu.sync_copy(va_hbm.at[pl.ds(base, GCHUNK)], rows_v)

        for j in range(NW_SUB):
            for k in range(0, WCH, 16):
                t = idx_v[pl.ds(j * WCH + k, 16)] - noff
                oob = (t < 0) | (t >= NH)
                idx2_v[j, pl.ds(k, 16)] = jnp.where(oob, NH, t)

        for j in range(NW_SUB):
            pltpu.sync_copy(rows_v.at[pl.ds(j * WCH, WCH)],
                            acc_sh.at[idx2_v.at[j]], add=True)

    plsc.subcore_barrier()

    @pl.when(tid < 5)
    def _():
        pltpu.sync_copy(acc_sh.at[pl.ds(tid * 1000, 1000)],
                        out_hbm.at[cid, pl.ds(tid * 1000, 1000)])


def _sc_scatter(va, src, zeros_nd):
    f = pl.kernel(
        _scatter_body,
        out_type=jax.ShapeDtypeStruct((SC_NC, NH, D), jnp.float32),
        mesh=_sc_mesh(),
        scratch_types=[
            pltpu.VMEM((GCHUNK,), jnp.int32),
            pltpu.VMEM((NW_SUB, WCH), jnp.int32),
            pltpu.VMEM((GCHUNK, D), jnp.float32),
            pltpu.VMEM_SHARED((NACC, D), jnp.float32),
        ],
        compiler_params=_SC_CP,
    )
    return f(va, src, zeros_nd)


def _qproj_body(atom_ref, wq_ref, q_ref):
    q_ref[...] = jnp.dot(atom_ref[...], wq_ref[...],
                         preferred_element_type=jnp.float32)


def _edge_attn_body(qs_ref, xd_ref, ef_ref, sh_ref,
                    wk1_ref, bk1_ref, wk2_ref, bk2_ref, attn_ref):
    ef = ef_ref[...]
    kh = jnp.maximum(jnp.dot(ef, wk1_ref[...],
                             preferred_element_type=jnp.float32)
                     + bk1_ref[...], 0.0)
    kw = jnp.dot(kh, wk2_ref[...],
                 preferred_element_type=jnp.float32) + bk2_ref[...]
    attn_ref[...] = jnp.sum(qs_ref[...] * xd_ref[...] * kw,
                            axis=1, keepdims=True) * sh_ref[...]


def _edge_va_body(xd_ref, ef_ref, sh_ref, al_ref,
                  wv1_ref, bv1_ref, wv2_ref, bv2_ref, va_ref):
    ef = ef_ref[...]
    vh = jnp.maximum(jnp.dot(ef, wv1_ref[...],
                             preferred_element_type=jnp.float32)
                     + bv1_ref[...], 0.0)
    vw = jnp.dot(vh, wv2_ref[...],
                 preferred_element_type=jnp.float32) + bv2_ref[...]
    va_ref[...] = xd_ref[...] * (sh_ref[...] * al_ref[...]) * vw


def _node_update_body(atom_ref, ulo_ref, uhi_ref, gamma_ref, beta_ref, we_ref,
                      x_ref, sa_ref):
    x = atom_ref[...] + jnp.concatenate([ulo_ref[...], uhi_ref[...]], axis=0)
    mu = jnp.mean(x, axis=0, keepdims=True)
    xc = x - mu
    var = jnp.mean(xc * xc, axis=0, keepdims=True)
    xn = xc * jax.lax.rsqrt(var + 1e-5) * gamma_ref[...] + beta_ref[...]
    x_ref[...] = xn
    sa = jnp.dot(xn, we_ref[...], preferred_element_type=jnp.float32)
    sa_ref[...] = jnp.concatenate(
        [sa, jnp.zeros((sa.shape[0], D - H), jnp.float32)], axis=1)


def _edge_update_body(sd_ref, ss_ref, ef_ref,
                      we1a_ref, we1b_ref, we1c_ref, be1_ref,
                      we2_ref, be2_ref, we3_ref, be3_ref,
                      lng_ref, lnb_ref, e_ref):
    ef = ef_ref[...]
    h = (jnp.dot(sd_ref[:, 0:H], we1a_ref[...], preferred_element_type=jnp.float32)
         + jnp.dot(ss_ref[:, 0:H], we1b_ref[...], preferred_element_type=jnp.float32)
         + jnp.dot(ef, we1c_ref[...], preferred_element_type=jnp.float32)
         + be1_ref[...])
    h = jnp.maximum(h, 0.0)
    h = jnp.maximum(jnp.dot(h, we2_ref[...],
                            preferred_element_type=jnp.float32)
                    + be2_ref[...], 0.0)
    h = jnp.dot(h, we3_ref[...], preferred_element_type=jnp.float32) + be3_ref[...]
    e = ef + h
    mu = jnp.mean(e, axis=1, keepdims=True)
    ec = e - mu
    var = jnp.mean(ec * ec, axis=1, keepdims=True)
    e_ref[...] = ec * jax.lax.rsqrt(var + 1e-5) * lng_ref[...] + lnb_ref[...]


def _full_spec(shape):
    return pl.BlockSpec(shape, lambda *_: tuple(0 for _ in shape))


def kernel(atom_features, edge_features, edge_sh, edge_index,
           W_q, Wk1, bk1, Wk2, bk2, Wv1, bv1, Wv2, bv2,
           bn_gamma, bn_beta, W_e, We1, be1, We2, be2, We3, be3,
           ln_gamma, ln_beta, interpret=False):
    dst = edge_index[0]
    src = edge_index[1]

    qfull = pl.pallas_call(
        _qproj_body,
        out_shape=jax.ShapeDtypeStruct((N, D), jnp.float32),
        interpret=interpret,
    )(atom_features, W_q)

    if interpret:
        qs = jnp.take(qfull, src, axis=0)
        xd = jnp.take(atom_features, dst, axis=0)
    else:
        qs, xd = _sc_gather2(qfull, atom_features, src, dst)

    eb = lambda i: (i, 0)
    attn2 = pl.pallas_call(
        _edge_attn_body,
        grid=(NEB,),
        in_specs=[
            pl.BlockSpec((EBLK, D), eb),
            pl.BlockSpec((EBLK, D), eb),
            pl.BlockSpec((EBLK, H), eb),
            pl.BlockSpec((EBLK, 1), eb),
            _full_spec((H, H)), _full_spec((1, H)),
            _full_spec((H, D)), _full_spec((1, D)),
        ],
        out_specs=pl.BlockSpec((EBLK, 1), eb),
        out_shape=jax.ShapeDtypeStruct((E, 1), jnp.float32),
        interpret=interpret,
    )(qs, xd, edge_features, edge_sh,
      Wk1, bk1.reshape(1, H), Wk2, bk2.reshape(1, D))
    attn = attn2[:, 0]

    if interpret:
        m = jax.ops.segment_max(attn, src, num_segments=N)
        m = jnp.where(jnp.isfinite(m), m, 0.0)
        ex = jnp.exp(attn - m[src])
        s = jax.ops.segment_sum(ex, src, num_segments=N)
        alpha = ex / (s[src] + 1e-16)
    else:
        alpha = _sc_softmax(attn, src)

    va = pl.pallas_call(
        _edge_va_body,
        grid=(NEB,),
        in_specs=[
            pl.BlockSpec((EBLK, D), eb),
            pl.BlockSpec((EBLK, H), eb),
            pl.BlockSpec((EBLK, 1), eb),
            pl.BlockSpec((EBLK, 1), eb),
            _full_spec((H, H)), _full_spec((1, H)),
            _full_spec((H, D)), _full_spec((1, D)),
        ],
        out_specs=pl.BlockSpec((EBLK, D), eb),
        out_shape=jax.ShapeDtypeStruct((E, D), jnp.float32),
        interpret=interpret,
    )(xd, edge_features, edge_sh, alpha.reshape(E, 1),
      Wv1, bv1.reshape(1, H), Wv2, bv2.reshape(1, D))

    if interpret:
        upd = jax.ops.segment_sum(va, src, num_segments=N)
        upd_a, upd_b = upd[0:NH], upd[NH:N]
    else:
        zeros_nd = jnp.zeros((NACC, D), jnp.float32)
        updp = _sc_scatter(va, src, zeros_nd)
        upd_a, upd_b = updp[0], updp[1]

    x, sa = pl.pallas_call(
        _node_update_body,
        out_shape=[
            jax.ShapeDtypeStruct((N, D), jnp.float32),
            jax.ShapeDtypeStruct((N, D), jnp.float32),
        ],
        interpret=interpret,
    )(atom_features, upd_a, upd_b,
      bn_gamma.reshape(1, D), bn_beta.reshape(1, D), W_e)

    if interpret:
        sd = jnp.take(sa, dst, axis=0)
        ss = jnp.take(sa, src, axis=0)
    else:
        ss, sd = _sc_gather2(sa, sa, src, dst)

    e = pl.pallas_call(
        _edge_update_body,
        grid=(NEB,),
        in_specs=[
            pl.BlockSpec((EBLK, D), eb),
            pl.BlockSpec((EBLK, D), eb),
            pl.BlockSpec((EBLK, H), eb),
            _full_spec((H, H)), _full_spec((H, H)), _full_spec((H, H)),
            _full_spec((1, H)),
            _full_spec((H, H)), _full_spec((1, H)),
            _full_spec((H, H)), _full_spec((1, H)),
            _full_spec((1, H)), _full_spec((1, H)),
        ],
        out_specs=pl.BlockSpec((EBLK, H), eb),
        out_shape=jax.ShapeDtypeStruct((E, H), jnp.float32),
        interpret=interpret,
    )(sd, ss, edge_features,
      We1[0:H], We1[H:2 * H], We1[2 * H:3 * H], be1.reshape(1, H),
      We2, be2.reshape(1, H), We3, be3.reshape(1, H),
      ln_gamma.reshape(1, H), ln_beta.reshape(1, H))

    return (x, e)


# speedup vs baseline: 4.4607x; 1.0004x over previous
import jax
import jax.numpy as jnp
from jax import lax
from jax.experimental import pallas as pl
from jax.experimental.pallas import tpu as pltpu
from jax.experimental.pallas import tpu_sc as plsc

N = 10000
E = 320000
D = 128
H = 16

EBLK = 6400
NEB = E // EBLK

SC_NC = 2
SC_NS = 16
SC_NW = SC_NC * SC_NS
GCHUNK = 400
EPW = E // SC_NW
NSTEP = EPW // GCHUNK
GC2 = 200
NSTEP2 = EPW // GC2


def _sc_mesh():
    return plsc.VectorSubcoreMesh(core_axis_name="c", subcore_axis_name="s")


_SC_CP = pltpu.CompilerParams(needs_layout_passes=False)


def _gather2_body(qtab_hbm, xtab_hbm, src_hbm, dst_hbm, qs_hbm, xd_hbm,
                  sidx_v, didx_v, qrows_v, xrows_v, sem_g, sem_w):
    wid = lax.axis_index("s") * SC_NC + lax.axis_index("c")
    base0 = wid * EPW

    pltpu.sync_copy(src_hbm.at[pl.ds(base0, EPW)], sidx_v)
    pltpu.sync_copy(dst_hbm.at[pl.ds(base0, EPW)], didx_v)

    def start_gather(c, b):
        off = c * GC2
        pltpu.async_copy(qtab_hbm.at[sidx_v.at[pl.ds(off, GC2)]],
                         qrows_v.at[b], sem_g.at[b])
        pltpu.async_copy(xtab_hbm.at[didx_v.at[pl.ds(off, GC2)]],
                         xrows_v.at[b], sem_g.at[b])

    def wait_gather(b):
        pltpu.make_async_copy(qtab_hbm.at[sidx_v.at[pl.ds(0, GC2)]],
                              qrows_v.at[b], sem_g.at[b]).wait()
        pltpu.make_async_copy(xtab_hbm.at[didx_v.at[pl.ds(0, GC2)]],
                              xrows_v.at[b], sem_g.at[b]).wait()

    def start_write(c, b):
        base = base0 + c * GC2
        pltpu.async_copy(qrows_v.at[b], qs_hbm.at[pl.ds(base, GC2)],
                         sem_w.at[b])
        pltpu.async_copy(xrows_v.at[b], xd_hbm.at[pl.ds(base, GC2)],
                         sem_w.at[b])

    def wait_write(b):
        pltpu.make_async_copy(qrows_v.at[b],
                              qs_hbm.at[pl.ds(base0, GC2)],
                              sem_w.at[b]).wait()
        pltpu.make_async_copy(xrows_v.at[b],
                              xd_hbm.at[pl.ds(base0, GC2)],
                              sem_w.at[b]).wait()

    @pl.loop(0, NSTEP2, step=2)
    def _(c):
        for b in range(2):
            cc = c + b

            @pl.when(cc >= 2)
            def _():
                wait_write(b)

            start_gather(cc, b)

            @pl.when(cc >= 1)
            def _():
                wait_gather(1 - b)
                start_write(cc - 1, 1 - b)

    last = NSTEP2 - 1
    lb = last % 2
    wait_gather(lb)
    start_write(last, lb)
    wait_write(1 - lb)
    wait_write(lb)


def _sc_gather2(qtab, xtab, src, dst):
    f = pl.kernel(
        _gather2_body,
        out_type=[
            jax.ShapeDtypeStruct((E, D), jnp.float32),
            jax.ShapeDtypeStruct((E, D), jnp.float32),
        ],
        mesh=_sc_mesh(),
        scratch_types=[
            pltpu.VMEM((EPW,), jnp.int32),
            pltpu.VMEM((EPW,), jnp.int32),
            pltpu.VMEM((2, GC2, D), jnp.float32),
            pltpu.VMEM((2, GC2, D), jnp.float32),
            pltpu.SemaphoreType.DMA((2,)),
            pltpu.SemaphoreType.DMA((2,)),
        ],
    )
    return f(qtab, xtab, src, dst)


NPAD = 10240
EPT = E // SC_NS
NROW = NPAD // SC_NS


def _softmax_body(attn_hbm, src_hbm, alpha_hbm,
                  m_v, s_v, attn_v, idx_v, ex_v, tmp_v, acc_v,
                  stage_sh, final_sh):
    cid = lax.axis_index("c")
    tid = lax.axis_index("s")
    ebase = tid * EPT
    rbase = tid * NROW

    @pl.when(cid == 0)
    def _():
        @pl.loop(0, NPAD, step=16)
        def _(i):
            m_v[pl.ds(i, 16)] = jnp.full((16,), -3e38, jnp.float32)

        pltpu.sync_copy(attn_hbm.at[pl.ds(ebase, EPT)], attn_v)
        pltpu.sync_copy(src_hbm.at[pl.ds(ebase, EPT)], idx_v)

        @pl.loop(0, EPT, step=16)
        def _(i):
            iv = idx_v[pl.ds(i, 16)]
            av = attn_v[pl.ds(i, 16)]
            mg = plsc.load_gather(m_v, [iv])
            plsc.store_scatter(m_v, [iv], jnp.maximum(mg, av))

        pltpu.sync_copy(m_v, stage_sh.at[tid])

    plsc.subcore_barrier()

    @pl.when(cid == 0)
    def _():
        pltpu.sync_copy(stage_sh.at[0, pl.ds(rbase, NROW)], acc_v)

        @pl.loop(1, SC_NS)
        def _(j):
            pltpu.sync_copy(stage_sh.at[j, pl.ds(rbase, NROW)], tmp_v)

            @pl.loop(0, NROW, step=16)
            def _(i):
                acc_v[pl.ds(i, 16)] = jnp.maximum(acc_v[pl.ds(i, 16)],
                                                  tmp_v[pl.ds(i, 16)])

        pltpu.sync_copy(acc_v, final_sh.at[pl.ds(rbase, NROW)])

    plsc.subcore_barrier()

    @pl.when(cid == 0)
    def _():
        pltpu.sync_copy(final_sh, m_v)

        @pl.loop(0, NPAD, step=16)
        def _(i):
            s_v[pl.ds(i, 16)] = jnp.zeros((16,), jnp.float32)

        @pl.loop(0, EPT, step=16)
        def _(i):
            iv = idx_v[pl.ds(i, 16)]
            av = attn_v[pl.ds(i, 16)]
            mg = plsc.load_gather(m_v, [iv])
            ex = jnp.exp(av - mg)
            ex_v[pl.ds(i, 16)] = ex
            plsc.addupdate_scatter(s_v, [iv], ex)

        pltpu.sync_copy(s_v, stage_sh.at[tid])

    plsc.subcore_barrier()

    @pl.when(cid == 0)
    def _():
        pltpu.sync_copy(stage_sh.at[0, pl.ds(rbase, NROW)], acc_v)

        @pl.loop(1, SC_NS)
        def _(j):
            pltpu.sync_copy(stage_sh.at[j, pl.ds(rbase, NROW)], tmp_v)

            @pl.loop(0, NROW, step=16)
            def _(i):
                acc_v[pl.ds(i, 16)] = acc_v[pl.ds(i, 16)] + tmp_v[pl.ds(i, 16)]

        @pl.loop(0, NROW, step=16)
        def _(i):
            acc_v[pl.ds(i, 16)] = 1.0 / (acc_v[pl.ds(i, 16)] + 1e-16)

        pltpu.sync_copy(acc_v, final_sh.at[pl.ds(rbase, NROW)])

    plsc.subcore_barrier()

    @pl.when(cid == 0)
    def _():
        pltpu.sync_copy(final_sh, m_v)

        @pl.loop(0, EPT, step=16)
        def _(i):
            iv = idx_v[pl.ds(i, 16)]
            rg = plsc.load_gather(m_v, [iv])
            ex_v[pl.ds(i, 16)] = ex_v[pl.ds(i, 16)] * rg

        pltpu.sync_copy(ex_v, alpha_hbm.at[pl.ds(ebase, EPT)])


def _sc_softmax(attn, src):
    f = pl.kernel(
        _softmax_body,
        out_type=jax.ShapeDtypeStruct((E,), jnp.float32),
        mesh=_sc_mesh(),
        scratch_types=[
            pltpu.VMEM((NPAD,), jnp.float32),
            pltpu.VMEM((NPAD,), jnp.float32),
            pltpu.VMEM((EPT,), jnp.float32),
            pltpu.VMEM((EPT,), jnp.int32),
            pltpu.VMEM((EPT,), jnp.float32),
            pltpu.VMEM((NROW,), jnp.float32),
            pltpu.VMEM((NROW,), jnp.float32),
            pltpu.VMEM_SHARED((SC_NS, NPAD), jnp.float32),
            pltpu.VMEM_SHARED((NPAD,), jnp.float32),
        ],
        compiler_params=_SC_CP,
    )
    return f(attn, src)


NH = N // SC_NC
NACC = NH + 8
EPT_S = E // SC_NS
NSTEP_S = EPT_S // GCHUNK
WCH = 80
NW_SUB = GCHUNK // WCH


def _scatter_body(va_hbm, src_hbm, zeros_hbm, out_hbm,
                  idx_v, idx2_v, rows_v, acc_sh):
    cid = lax.axis_index("c")
    tid = lax.axis_index("s")
    base0 = tid * EPT_S
    noff = cid * NH

    @pl.when(tid == 0)
    def _():
        pltpu.sync_copy(zeros_hbm, acc_sh)

    plsc.subcore_barrier()

    @pl.loop(0, NSTEP_S)
    def _(c):
        base = base0 + c * GCHUNK
        pltpu.sync_copy(src_hbm.at[pl.ds(base, GCHUNK)], idx_v)
        pltpu.sync_copy(va_hbm.at[pl.ds(base, GCHUNK)], rows_v)

        for j in range(NW_SUB):
            for k in range(0, WCH, 16):
                t = idx_v[pl.ds(j * WCH + k, 16)] - noff
                oob = (t < 0) | (t >= NH)
                idx2_v[j, pl.ds(k, 16)] = jnp.where(oob, NH, t)

        for j in range(NW_SUB):
            pltpu.sync_copy(rows_v.at[pl.ds(j * WCH, WCH)],
                            acc_sh.at[idx2_v.at[j]], add=True)

    plsc.subcore_barrier()

    @pl.when(tid < 5)
    def _():
        pltpu.sync_copy(acc_sh.at[pl.ds(tid * 1000, 1000)],
                        out_hbm.at[cid, pl.ds(tid * 1000, 1000)])


def _sc_scatter(va, src, zeros_nd):
    f = pl.kernel(
        _scatter_body,
        out_type=jax.ShapeDtypeStruct((SC_NC, NH, D), jnp.float32),
        mesh=_sc_mesh(),
        scratch_types=[
            pltpu.VMEM((GCHUNK,), jnp.int32),
            pltpu.VMEM((NW_SUB, WCH), jnp.int32),
            pltpu.VMEM((GCHUNK, D), jnp.float32),
            pltpu.VMEM_SHARED((NACC, D), jnp.float32),
        ],
        compiler_params=_SC_CP,
    )
    return f(va, src, zeros_nd)


def _qproj_body(atom_ref, wq_ref, q_ref):
    q_ref[...] = jnp.dot(atom_ref[...], wq_ref[...],
                         preferred_element_type=jnp.float32)


def _edge_attn_body(qs_ref, xd_ref, ef_ref, sh_ref,
                    wk1_ref, bk1_ref, wk2_ref, bk2_ref, attn_ref):
    ef = ef_ref[...]
    kh = jnp.maximum(jnp.dot(ef, wk1_ref[...],
                             preferred_element_type=jnp.float32)
                     + bk1_ref[...], 0.0)
    kw = jnp.dot(kh, wk2_ref[...],
                 preferred_element_type=jnp.float32) + bk2_ref[...]
    attn_ref[...] = jnp.sum(qs_ref[...] * xd_ref[...] * kw,
                            axis=1, keepdims=True) * sh_ref[...]


def _edge_va_body(xd_ref, ef_ref, sh_ref, al_ref,
                  wv1_ref, bv1_ref, wv2_ref, bv2_ref, va_ref):
    ef = ef_ref[...]
    vh = jnp.maximum(jnp.dot(ef, wv1_ref[...],
                             preferred_element_type=jnp.float32)
                     + bv1_ref[...], 0.0)
    vw = jnp.dot(vh, wv2_ref[...],
                 preferred_element_type=jnp.float32) + bv2_ref[...]
    va_ref[...] = xd_ref[...] * (sh_ref[...] * al_ref[...]) * vw


def _node_update_body(atom_ref, ulo_ref, uhi_ref, gamma_ref, beta_ref, we_ref,
                      x_ref, sa_ref):
    x = atom_ref[...] + jnp.concatenate([ulo_ref[...], uhi_ref[...]], axis=0)
    mu = jnp.mean(x, axis=0, keepdims=True)
    xc = x - mu
    var = jnp.mean(xc * xc, axis=0, keepdims=True)
    xn = xc * jax.lax.rsqrt(var + 1e-5) * gamma_ref[...] + beta_ref[...]
    x_ref[...] = xn
    sa = jnp.dot(xn, we_ref[...], preferred_element_type=jnp.float32)
    sa_ref[...] = jnp.concatenate(
        [sa, jnp.zeros((sa.shape[0], D - H), jnp.float32)], axis=1)


def _edge_update_body(sd_ref, ss_ref, ef_ref,
                      we1a_ref, we1b_ref, we1c_ref, be1_ref,
                      we2_ref, be2_ref, we3_ref, be3_ref,
                      lng_ref, lnb_ref, e_ref):
    ef = ef_ref[...]
    h = (jnp.dot(sd_ref[:, 0:H], we1a_ref[...], preferred_element_type=jnp.float32)
         + jnp.dot(ss_ref[:, 0:H], we1b_ref[...], preferred_element_type=jnp.float32)
         + jnp.dot(ef, we1c_ref[...], preferred_element_type=jnp.float32)
         + be1_ref[...])
    h = jnp.maximum(h, 0.0)
    h = jnp.maximum(jnp.dot(h, we2_ref[...],
                            preferred_element_type=jnp.float32)
                    + be2_ref[...], 0.0)
    h = jnp.dot(h, we3_ref[...], preferred_element_type=jnp.float32) + be3_ref[...]
    e = ef + h
    mu = jnp.mean(e, axis=1, keepdims=True)
    ec = e - mu
    var = jnp.mean(ec * ec, axis=1, keepdims=True)
    e_ref[...] = ec * jax.lax.rsqrt(var + 1e-5) * lng_ref[...] + lnb_ref[...]


def _full_spec(shape):
    return pl.BlockSpec(shape, lambda *_: tuple(0 for _ in shape))


def kernel(atom_features, edge_features, edge_sh, edge_index,
           W_q, Wk1, bk1, Wk2, bk2, Wv1, bv1, Wv2, bv2,
           bn_gamma, bn_beta, W_e, We1, be1, We2, be2, We3, be3,
           ln_gamma, ln_beta):
    dst = edge_index[0]
    src = edge_index[1]

    qfull = pl.pallas_call(
        _qproj_body,
        out_shape=jax.ShapeDtypeStruct((N, D), jnp.float32),
    )(atom_features, W_q)

    qs, xd = _sc_gather2(qfull, atom_features, src, dst)

    eb = lambda i: (i, 0)
    attn2 = pl.pallas_call(
        _edge_attn_body,
        grid=(NEB,),
        in_specs=[
            pl.BlockSpec((EBLK, D), eb),
            pl.BlockSpec((EBLK, D), eb),
            pl.BlockSpec((EBLK, H), eb),
            pl.BlockSpec((EBLK, 1), eb),
            _full_spec((H, H)), _full_spec((1, H)),
            _full_spec((H, D)), _full_spec((1, D)),
        ],
        out_specs=pl.BlockSpec((EBLK, 1), eb),
        out_shape=jax.ShapeDtypeStruct((E, 1), jnp.float32),
    )(qs, xd, edge_features, edge_sh,
      Wk1, bk1.reshape(1, H), Wk2, bk2.reshape(1, D))
    attn = attn2[:, 0]

    alpha = _sc_softmax(attn, src)

    va = pl.pallas_call(
        _edge_va_body,
        grid=(NEB,),
        in_specs=[
            pl.BlockSpec((EBLK, D), eb),
            pl.BlockSpec((EBLK, H), eb),
            pl.BlockSpec((EBLK, 1), eb),
            pl.BlockSpec((EBLK, 1), eb),
            _full_spec((H, H)), _full_spec((1, H)),
            _full_spec((H, D)), _full_spec((1, D)),
        ],
        out_specs=pl.BlockSpec((EBLK, D), eb),
        out_shape=jax.ShapeDtypeStruct((E, D), jnp.float32),
    )(xd, edge_features, edge_sh, alpha.reshape(E, 1),
      Wv1, bv1.reshape(1, H), Wv2, bv2.reshape(1, D))

    zeros_nd = jnp.zeros((NACC, D), jnp.float32)
    updp = _sc_scatter(va, src, zeros_nd)
    upd_a, upd_b = updp[0], updp[1]

    x, sa = pl.pallas_call(
        _node_update_body,
        out_shape=[
            jax.ShapeDtypeStruct((N, D), jnp.float32),
            jax.ShapeDtypeStruct((N, D), jnp.float32),
        ],
    )(atom_features, upd_a, upd_b,
      bn_gamma.reshape(1, D), bn_beta.reshape(1, D), W_e)

    ss, sd = _sc_gather2(sa, sa, src, dst)

    e = pl.pallas_call(
        _edge_update_body,
        grid=(NEB,),
        in_specs=[
            pl.BlockSpec((EBLK, D), eb),
            pl.BlockSpec((EBLK, D), eb),
            pl.BlockSpec((EBLK, H), eb),
            _full_spec((H, H)), _full_spec((H, H)), _full_spec((H, H)),
            _full_spec((1, H)),
            _full_spec((H, H)), _full_spec((1, H)),
            _full_spec((H, H)), _full_spec((1, H)),
            _full_spec((1, H)), _full_spec((1, H)),
        ],
        out_specs=pl.BlockSpec((EBLK, H), eb),
        out_shape=jax.ShapeDtypeStruct((E, H), jnp.float32),
    )(sd, ss, edge_features,
      We1[0:H], We1[H:2 * H], We1[2 * H:3 * H], be1.reshape(1, H),
      We2, be2.reshape(1, H), We3, be3.reshape(1, H),
      ln_gamma.reshape(1, H), ln_beta.reshape(1, H))

    return (x, e)
